# Initial kernel scaffold; baseline (speedup 1.0000x reference)
import jax
import jax.numpy as jnp
from jax.experimental import pallas as pl


def kernel(all_embed, relation_emb, inter_val, edge_index, edge_type, inter_row, inter_col, users, pos_items, neg_items):
    raise NotImplementedError("write your pallas kernel here")


# speedup vs baseline: 1.5455x; 1.5455x over previous
import functools

import jax
import jax.numpy as jnp
from jax import lax
from jax.experimental import pallas as pl
from jax.experimental.pallas import tpu as pltpu
from jax.experimental.pallas import tpu_sc as plsc

N_USERS = 2000
N_ENTITIES = 8000
DIM = 256
N_EDGES = 160000
NNZ = 64000
HOPS = 3
BATCH = 4096
DECAY = 1e-4

NC = 2
NS = 16
NW = NC * NS
L = 16
C = 128

_MESH = plsc.VectorSubcoreMesh(core_axis_name="c", subcore_axis_name="s")


def _zero_rows(zbuf, zrows):
    def body(i, carry):
        for d in range(DIM // L):
            zbuf[i, pl.ds(d * L, L)] = jnp.zeros((L,), jnp.float32)
        return carry
    lax.fori_loop(0, zrows, body, 0)


ZR = 200


def _seg_kernel_body(n_rows, n_edges, weighted, c_sz,
                     tbl_h, w_h, tail_h, head_h, type_h, out_h,
                     tail_v, head_v, type_v, val_v, rows_v, rel_v, zbuf):
    cid = lax.axis_index("c")
    sid = lax.axis_index("s")
    wid = sid * NC + cid

    _zero_rows(zbuf, ZR)
    for u in range(n_rows // ZR):
        pltpu.sync_copy(zbuf, out_h.at[pl.ds(wid * n_rows + u * ZR, ZR)])

    n_chunks = n_edges // c_sz
    iters = (n_chunks + NW - 1) // NW

    def chunk(k, carry):
        ci = k * NW + wid

        @pl.when(ci < n_chunks)
        def _():
            base = ci * c_sz
            pltpu.sync_copy(tail_h.at[pl.ds(base, c_sz)], tail_v)
            pltpu.sync_copy(head_h.at[pl.ds(base, c_sz)], head_v)
            pltpu.sync_copy(tbl_h.at[tail_v], rows_v)
            off = wid * n_rows
            for g in range(c_sz // L):
                sl = pl.ds(g * L, L)
                head_v[sl] = head_v[sl] + off
            if weighted:
                pltpu.sync_copy(type_h.at[pl.ds(base, c_sz)], type_v)
                pltpu.sync_copy(w_h.at[type_v], rel_v)
            else:
                pltpu.sync_copy(w_h.at[pl.ds(base, c_sz)], rel_v)

            def mul(i, carry2):
                for d in range(DIM // L):
                    sl = pl.ds(d * L, L)
                    rows_v[i, sl] = rows_v[i, sl] * rel_v[i, sl]
                return carry2

            lax.fori_loop(0, c_sz, mul, 0)
            pltpu.sync_copy(rows_v, out_h.at[head_v], add=True)
        return carry

    lax.fori_loop(0, iters, chunk, 0)


def _make_seg_kernel(n_rows, n_edges, weighted, c_sz):
    body = functools.partial(_seg_kernel_body, n_rows, n_edges, weighted, c_sz)
    return pl.kernel(
        body,
        out_type=jax.ShapeDtypeStruct((NW * n_rows, DIM), jnp.float32),
        mesh=_MESH,
        scratch_types=[
            pltpu.VMEM((c_sz,), jnp.int32),
            pltpu.VMEM((c_sz,), jnp.int32),
            pltpu.VMEM((c_sz,), jnp.int32),
            pltpu.VMEM((c_sz,), jnp.float32),
            pltpu.VMEM((c_sz, DIM), jnp.float32),
            pltpu.VMEM((c_sz, DIM), jnp.float32),
            pltpu.VMEM((ZR, DIM), jnp.float32),
        ],
    )


_hop_sc = _make_seg_kernel(N_ENTITIES, N_EDGES, True, C)
_user_sc = _make_seg_kernel(N_USERS, NNZ, False, C)


def _gather_body(ures_h, eres_h, u_h, p_h, n_h, oue, ope, one, idx_v, rows_v):
    cid = lax.axis_index("c")
    sid = lax.axis_index("s")
    wid = sid * NC + cid
    bpt = BATCH // NW
    base = wid * bpt
    for src_idx, out_h, tbl_h in ((u_h, oue, ures_h), (p_h, ope, eres_h), (n_h, one, eres_h)):
        pltpu.sync_copy(src_idx.at[pl.ds(base, bpt)], idx_v)
        pltpu.sync_copy(tbl_h.at[idx_v], rows_v)
        pltpu.sync_copy(rows_v, out_h.at[pl.ds(base, bpt)])


_gather_sc = pl.kernel(
    _gather_body,
    out_type=[jax.ShapeDtypeStruct((BATCH, DIM), jnp.float32)] * 3,
    mesh=_MESH,
    scratch_types=[
        pltpu.VMEM((BATCH // NW,), jnp.int32),
        pltpu.VMEM((BATCH // NW, DIM), jnp.float32),
    ],
)


def _add_body(a_ref, b_ref, o_ref):
    o_ref[...] = a_ref[...] + b_ref[...]


def _add_tc(a, b):
    n = a.shape[0]
    br = 1000
    return pl.pallas_call(
        _add_body,
        grid=(n // br,),
        in_specs=[pl.BlockSpec((br, DIM), lambda i: (i, 0))] * 2,
        out_specs=pl.BlockSpec((br, DIM), lambda i: (i, 0)),
        out_shape=jax.ShapeDtypeStruct((n, DIM), jnp.float32),
    )(a, b)


def _combine_body(p_ref, res_ref, e_ref, res2_ref):
    e = jnp.sum(p_ref[...], axis=0)
    e_ref[...] = e
    res2_ref[...] = res_ref[...] + e


def _combine_tc(p, res):
    n = res.shape[0]
    br = 400
    grid = n // br
    return pl.pallas_call(
        _combine_body,
        grid=(grid,),
        in_specs=[
            pl.BlockSpec((NW, br, DIM), lambda i: (0, i, 0)),
            pl.BlockSpec((br, DIM), lambda i: (i, 0)),
        ],
        out_specs=[
            pl.BlockSpec((br, DIM), lambda i: (i, 0)),
            pl.BlockSpec((br, DIM), lambda i: (i, 0)),
        ],
        out_shape=[jax.ShapeDtypeStruct((n, DIM), jnp.float32)] * 2,
    )(p, res)


def _loss_body(u_ref, p_ref, n_ref, loss_ref, mf_ref, emb_ref):
    u = u_ref[...]
    p = p_ref[...]
    n = n_ref[...]
    pos_s = jnp.sum(u * p, axis=1, keepdims=True)
    neg_s = jnp.sum(u * n, axis=1, keepdims=True)
    x = pos_s - neg_s
    mf = -jnp.mean(jax.nn.log_sigmoid(x))
    reg = 0.5 * (jnp.sum(u * u) + jnp.sum(p * p) + jnp.sum(n * n))
    emb = DECAY * reg / BATCH
    mf_ref[0, 0] = mf
    emb_ref[0, 0] = emb
    loss_ref[0, 0] = mf + emb


_loss_tc = pl.pallas_call(
    _loss_body,
    out_specs=[pl.BlockSpec(memory_space=pltpu.SMEM)] * 3,
    out_shape=[jax.ShapeDtypeStruct((1, 1), jnp.float32)] * 3,
)


def kernel(all_embed, relation_emb, inter_val, edge_index, edge_type,
           inter_row, inter_col, users, pos_items, neg_items):
    u0 = all_embed[:N_USERS]
    e0 = all_embed[N_USERS:]
    tail = edge_index[1].astype(jnp.int32)
    head = edge_index[0].astype(jnp.int32)
    etype = (edge_type - 1).astype(jnp.int32)
    icol = inter_col.astype(jnp.int32)
    irow = inter_row.astype(jnp.int32)
    rel = relation_emb.astype(jnp.float32)
    val = inter_val.astype(jnp.float32)

    emb = e0
    res = e0
    s2 = None
    for hop in range(HOPS):
        part = _hop_sc(emb, rel, tail, head, etype)
        emb, res = _combine_tc(part.reshape(NW, N_ENTITIES, DIM), res)
        if hop == HOPS - 2:
            s2 = res

    val_rows = jnp.broadcast_to(val[:, None], (NNZ, DIM))
    upart = _user_sc(s2, val_rows, icol, irow, icol)
    _, user_res = _combine_tc(upart.reshape(NW, N_USERS, DIM), u0)

    u_e, pos_e, neg_e = _gather_sc(
        user_res, res,
        users.astype(jnp.int32), pos_items.astype(jnp.int32),
        neg_items.astype(jnp.int32))

    loss, mf, emb_l = _loss_tc(u_e, pos_e, neg_e)
    return (loss[0, 0], mf[0, 0], emb_l[0, 0])
